# Initial kernel scaffold; baseline (speedup 1.0000x reference)
import jax
import jax.numpy as jnp
from jax.experimental import pallas as pl


def kernel(entity_emb, relation_emb, pos_h, pos_r, pos_t, neg_h, neg_r, neg_t):
    raise NotImplementedError("write your pallas kernel here")


# speedup vs baseline: 1.2574x; 1.2574x over previous
import functools

import jax
import jax.numpy as jnp
from jax import lax
from jax.experimental import pallas as pl
from jax.experimental.pallas import tpu as pltpu
from jax.experimental.pallas import tpu_sc as plsc

_EMB_DIM = 128
_BATCH = 16384
_PI = 3.141592653589793
_EMB_RANGE = (6.0 + 2.0) / _EMB_DIM
_PHASE_SCALE = _PI / _EMB_RANGE

_info = plsc.get_sparse_core_info()
_NC = _info.num_cores
_NS = _info.num_subcores
_L = _info.num_lanes
_NW = _NC * _NS

_TOTAL = 2 * _BATCH
_PER_W = _TOTAL // _NW
_C = 64
_NCHUNK = _PER_W // _C
_G = _C // _L


def _trig_body(r_ref, c_ref, s_ref):
    ph = r_ref[...] * _PHASE_SCALE
    c_ref[...] = jnp.cos(ph)
    s_ref[...] = jnp.sin(ph)


def _sqrt(m):
    m = m + 1e-35
    yi = plsc.bitcast(m, jnp.int32)
    yi = 0x5F3759DF - (yi >> 1)
    g = plsc.bitcast(yi, jnp.float32)
    hm = m * 0.5
    g = g * (1.5 - hm * g * g)
    g = g * (1.5 - hm * g * g)
    g = g * (1.5 - hm * g * g)
    return m * g


@functools.partial(
    pl.kernel,
    out_type=jax.ShapeDtypeStruct((_TOTAL,), jnp.float32),
    mesh=plsc.VectorSubcoreMesh(core_axis_name="c", subcore_axis_name="s"),
    compiler_params=pltpu.CompilerParams(
        use_tc_tiling_on_sc=False, needs_layout_passes=False),
    scratch_types=[
        pltpu.VMEM((_C, 2 * _EMB_DIM), jnp.float32),
        pltpu.VMEM((_C, 2 * _EMB_DIM), jnp.float32),
        pltpu.VMEM((_C, 2 * _EMB_DIM), jnp.float32),
        pltpu.VMEM((_C, 2 * _EMB_DIM), jnp.float32),
        pltpu.VMEM((_C, _EMB_DIM), jnp.float32),
        pltpu.VMEM((_C, _EMB_DIM), jnp.float32),
        pltpu.VMEM((_C, _EMB_DIM), jnp.float32),
        pltpu.VMEM((_C, _EMB_DIM), jnp.float32),
        pltpu.VMEM((_C,), jnp.int32),
        pltpu.VMEM((_C,), jnp.int32),
        pltpu.VMEM((_C,), jnp.int32),
        pltpu.VMEM((_C,), jnp.int32),
        pltpu.VMEM((_C,), jnp.int32),
        pltpu.VMEM((_C,), jnp.int32),
        pltpu.VMEM((_PER_W,), jnp.float32),
        pltpu.SemaphoreType.DMA,
        pltpu.SemaphoreType.DMA,
    ],
)
def _sc_score(ent, cost, sint, hh, rr, tt, out,
              eh0, eh1, et0, et1, cc0, cc1, ss0, ss1,
              hi0, hi1, ri0, ri1, ti0, ti1, ob, sem0, sem1):
    wid = lax.axis_index("s") * _NC + lax.axis_index("c")
    base = pl.multiple_of(wid * _PER_W, _PER_W)
    ehs = (eh0, eh1)
    ets = (et0, et1)
    ccs = (cc0, cc1)
    sss = (ss0, ss1)
    his = (hi0, hi1)
    ris = (ri0, ri1)
    tis = (ti0, ti1)
    sems = (sem0, sem1)

    def fire(g, b):
        off = pl.multiple_of(base + g * _C, _C)
        pltpu.sync_copy(hh.at[pl.ds(off, _C)], his[b])
        pltpu.sync_copy(rr.at[pl.ds(off, _C)], ris[b])
        pltpu.sync_copy(tt.at[pl.ds(off, _C)], tis[b])
        return (
            pltpu.async_copy(ent.at[his[b]], ehs[b], sems[b]),
            pltpu.async_copy(ent.at[tis[b]], ets[b], sems[b]),
            pltpu.async_copy(cost.at[ris[b]], ccs[b], sems[b]),
            pltpu.async_copy(sint.at[ris[b]], sss[b], sems[b]),
        )

    def compute(g, b):
        eh, et, cc, ss = ehs[b], ets[b], ccs[b], sss[b]
        rows = [lax.iota(jnp.int32, _L) + t * _L for t in range(_G)]
        zero_v = lax.iota(jnp.int32, _L) * 0

        def body(d, accs):
            col_c = zero_v + d
            col_re = zero_v + d * 2
            col_im = col_re + 1
            new = []
            for t in range(_G):
                reh = plsc.load_gather(eh, [rows[t], col_re])
                imh = plsc.load_gather(eh, [rows[t], col_im])
                ret = plsc.load_gather(et, [rows[t], col_re])
                imt = plsc.load_gather(et, [rows[t], col_im])
                cv = plsc.load_gather(cc, [rows[t], col_c])
                sv = plsc.load_gather(ss, [rows[t], col_c])
                rd = reh * cv - imh * sv - ret
                im = reh * sv + imh * cv - imt
                new.append(accs[t] + _sqrt(rd * rd + im * im))
            return tuple(new)

        accs = lax.fori_loop(
            0, _EMB_DIM, body,
            tuple(jnp.zeros((_L,), jnp.float32) for _ in range(_G)))
        for t in range(_G):
            ob[pl.ds(g * _C + t * _L, _L)] = accs[t]

    copies = {}
    copies[0] = fire(0, 0)
    for g in range(_NCHUNK):
        b = g & 1
        if g + 1 < _NCHUNK:
            copies[1 - b] = fire(g + 1, 1 - b)
        for cp in copies[b]:
            cp.wait()
        compute(g, b)
    pltpu.sync_copy(ob, out.at[pl.ds(base, _PER_W)])


def kernel(entity_emb, relation_emb, pos_h, pos_r, pos_t, neg_h, neg_r, neg_t):
    nrel, dim = relation_emb.shape
    trig = pl.pallas_call(
        _trig_body,
        out_shape=(
            jax.ShapeDtypeStruct((nrel, dim), jnp.float32),
            jax.ShapeDtypeStruct((nrel, dim), jnp.float32),
        ),
    )
    cos_t, sin_t = trig(relation_emb)
    hh = jnp.concatenate([pos_h, neg_h]).astype(jnp.int32)
    rr = jnp.concatenate([pos_r, neg_r]).astype(jnp.int32)
    tt = jnp.concatenate([pos_t, neg_t]).astype(jnp.int32)
    scores = _sc_score(entity_emb, cos_t, sin_t, hh, rr, tt)
    return scores[:_BATCH], scores[_BATCH:]
